# baseline (speedup 1.0000x reference)
import functools

import jax
import jax.numpy as jnp
from jax import lax
from jax.experimental import pallas as pl
from jax.experimental.pallas import tpu as pltpu
from jax.experimental.pallas import tpu_sc as plsc

BATCH = 16384
DIM = 64
NUM_CORES = 2
NUM_SUBCORES = 16
NUM_WORKERS = NUM_CORES * NUM_SUBCORES
B_PER_W = BATCH // NUM_WORKERS
UNROLL = 8


NSEM = 8
SEM_BLK = B_PER_W // NSEM


def _gather_body(idx_hbm, table_hbm, out_hbm, idx_vmem, rows_v, *sems):
    wid = lax.axis_index("s") * NUM_CORES + lax.axis_index("c")
    base = wid * B_PER_W
    pltpu.sync_copy(idx_hbm.at[pl.ds(base, B_PER_W)], idx_vmem)

    def issue(c, carry):
        vec = idx_vmem[pl.ds(c * 16, 16)]
        for j in range(16):
            i = c * 16 + j
            pltpu.async_copy(table_hbm.at[vec[j]], rows_v.at[i],
                             sems[j % NSEM])
        return carry

    lax.fori_loop(0, B_PER_W // 16, issue, 0, unroll=False)
    for s in range(NSEM):
        pltpu.make_async_copy(
            table_hbm.at[pl.ds(0, SEM_BLK)],
            rows_v.at[pl.ds(s * SEM_BLK, SEM_BLK)],
            sems[s],
        ).wait()
    pltpu.sync_copy(rows_v, out_hbm.at[pl.ds(base, B_PER_W)])


@jax.jit
def kernel(speaker_indices, weight):
    mesh = plsc.VectorSubcoreMesh(core_axis_name="c", subcore_axis_name="s")
    k = functools.partial(
        pl.kernel,
        mesh=mesh,
        out_type=jax.ShapeDtypeStruct((BATCH, DIM), jnp.float32),
        scratch_types=[
            pltpu.VMEM((B_PER_W,), jnp.int32),
            pltpu.VMEM((B_PER_W, DIM), jnp.float32),
        ] + [pltpu.SemaphoreType.DMA] * NSEM,
    )(_gather_body)
    return k(speaker_indices.astype(jnp.int32), weight)
